# Initial kernel scaffold; baseline (speedup 1.0000x reference)
import jax
import jax.numpy as jnp
from jax.experimental import pallas as pl


def kernel(x, edge_index, batch, W1, b1, W2, b2, W3, b3):
    raise NotImplementedError("write your pallas kernel here")


# speedup vs baseline: 23.3574x; 23.3574x over previous
import functools

import jax
import jax.numpy as jnp
from jax import lax
from jax.experimental import pallas as pl
from jax.experimental.pallas import tpu as pltpu
from jax.experimental.pallas import tpu_sc as plsc

N = 10000
E = 320000
NGRAPH = 64
NCLS = 16
NCORES = 2
NSUB = 16
NW = NCORES * NSUB
CHUNK = 80
NCHUNK = (E // NW) // CHUNK
ROWS_SUB = N // NSUB
DEGW = 16

_mesh = plsc.VectorSubcoreMesh(core_axis_name="c", subcore_axis_name="s")
_sc_params = pltpu.CompilerParams(use_tc_tiling_on_sc=False)


@functools.partial(
    pl.kernel,
    out_type=jax.ShapeDtypeStruct((NCORES, NSUB, ROWS_SUB, DEGW), jnp.float32),
    mesh=_mesh,
    compiler_params=_sc_params,
    scratch_types=[
        pltpu.VMEM((NCHUNK, CHUNK), jnp.int32),
        pltpu.VMEM((CHUNK, DEGW), jnp.float32),
        pltpu.VMEM_SHARED((N, DEGW), jnp.float32),
    ],
)
def _deg_kernel(dst_hbm, ones_hbm, zeros_hbm, out_hbm, dst_v, ones_v, acc):
    c = lax.axis_index("c")
    s = lax.axis_index("s")
    wid = c * NSUB + s
    pltpu.sync_copy(zeros_hbm, acc.at[pl.ds(s * ROWS_SUB, ROWS_SUB)])
    pltpu.sync_copy(ones_hbm, ones_v)
    pltpu.sync_copy(dst_hbm.at[wid], dst_v)
    plsc.subcore_barrier()

    def body(i, carry):
        pltpu.sync_copy(ones_v, acc.at[dst_v.at[i]], add=True)
        return carry

    lax.fori_loop(0, NCHUNK, body, 0)
    plsc.subcore_barrier()
    pltpu.sync_copy(acc.at[pl.ds(s * ROWS_SUB, ROWS_SUB)], out_hbm.at[c, s])


def _make_agg(F):
    @functools.partial(
        pl.kernel,
        out_type=jax.ShapeDtypeStruct((NCORES, NSUB, ROWS_SUB, F), jnp.float32),
        mesh=_mesh,
        compiler_params=_sc_params,
        scratch_types=[
            pltpu.VMEM((NCHUNK, CHUNK), jnp.int32),
            pltpu.VMEM((NCHUNK, CHUNK), jnp.int32),
            pltpu.VMEM((CHUNK, F), jnp.float32),
            pltpu.SemaphoreType.DMA,
            pltpu.VMEM_SHARED((N, F), jnp.float32),
        ],
    )
    def agg(hs_hbm, src_hbm, dst_hbm, zeros_hbm, out_hbm,
            src_v, dst_v, rows_v, sem, acc):
        c = lax.axis_index("c")
        s = lax.axis_index("s")
        wid = c * NSUB + s
        pltpu.sync_copy(zeros_hbm, acc.at[pl.ds(s * ROWS_SUB, ROWS_SUB)])
        pltpu.sync_copy(src_hbm.at[wid], src_v)
        pltpu.sync_copy(dst_hbm.at[wid], dst_v)
        plsc.subcore_barrier()

        def body(i, carry):
            pltpu.async_copy(hs_hbm.at[src_v.at[i]], rows_v, sem).wait()
            pltpu.sync_copy(rows_v, acc.at[dst_v.at[i]], add=True)
            return carry

        lax.fori_loop(0, NCHUNK, body, 0)
        plsc.subcore_barrier()
        pltpu.sync_copy(acc.at[pl.ds(s * ROWS_SUB, ROWS_SUB)], out_hbm.at[c, s])

    return agg


_AGG = {F: _make_agg(F) for F in (32, 64, 16)}


def _tc_head(parts_ref, x_ref, w_ref, hs_ref, dinv_ref):
    deg = (parts_ref[0] + parts_ref[1])[:, 0:1] + 1.0
    dinv = lax.rsqrt(deg)
    p = jnp.dot(x_ref[...], w_ref[...], preferred_element_type=jnp.float32)
    hs_ref[...] = p * dinv
    dinv_ref[...] = dinv


def _tc_mid(parts_ref, hs_ref, dinv_ref, b_ref, w_ref, out_ref):
    dinv = dinv_ref[...]
    h = dinv * (parts_ref[0] + parts_ref[1] + hs_ref[...]) + b_ref[...]
    h = jnp.maximum(h, 0.0)
    out_ref[...] = jnp.dot(h, w_ref[...],
                           preferred_element_type=jnp.float32) * dinv


def _tc_tail(parts_ref, hs_ref, dinv_ref, b_ref, batch_ref, out_ref):
    dinv = dinv_ref[...]
    h3 = dinv * (parts_ref[0] + parts_ref[1] + hs_ref[...]) + b_ref[...]
    gids = lax.broadcasted_iota(jnp.int32, (NGRAPH, N), 0)
    sel = (batch_ref[...] == gids).astype(jnp.float32)
    sums = jnp.dot(sel, h3, preferred_element_type=jnp.float32)
    counts = jnp.sum(sel, axis=1, keepdims=True)
    pooled = sums / jnp.maximum(counts, 1.0)
    m = jnp.max(pooled, axis=1, keepdims=True)
    lse = jnp.log(jnp.sum(jnp.exp(pooled - m), axis=1, keepdims=True)) + m
    out_ref[...] = pooled - lse


def _call_tc(body, out_shapes, *args):
    return pl.pallas_call(
        body,
        out_shape=out_shapes,
    )(*args)


@jax.jit
def kernel(x, edge_index, batch, W1, b1, W2, b2, W3, b3):
    src = edge_index[0].reshape(NW, NCHUNK, CHUNK)
    dst = edge_index[1].reshape(NW, NCHUNK, CHUNK)
    ones_deg = jnp.ones((CHUNK, DEGW), jnp.float32)
    zeros_deg = jnp.zeros((ROWS_SUB, DEGW), jnp.float32)

    deg_parts = _deg_kernel(dst, ones_deg, zeros_deg).reshape(NCORES, N, DEGW)

    hs1, dinv = _call_tc(
        _tc_head,
        [jax.ShapeDtypeStruct((N, 32), jnp.float32),
         jax.ShapeDtypeStruct((N, 1), jnp.float32)],
        deg_parts, x, W1)

    acc1 = _AGG[32](hs1, src, dst,
                    jnp.zeros((ROWS_SUB, 32), jnp.float32)).reshape(NCORES, N, 32)
    hs2 = _call_tc(
        _tc_mid, jax.ShapeDtypeStruct((N, 64), jnp.float32),
        acc1, hs1, dinv, b1.reshape(1, 32), W2)

    acc2 = _AGG[64](hs2, src, dst,
                    jnp.zeros((ROWS_SUB, 64), jnp.float32)).reshape(NCORES, N, 64)
    hs3 = _call_tc(
        _tc_mid, jax.ShapeDtypeStruct((N, NCLS), jnp.float32),
        acc2, hs2, dinv, b2.reshape(1, 64), W3)

    acc3 = _AGG[16](hs3, src, dst,
                    jnp.zeros((ROWS_SUB, NCLS), jnp.float32)).reshape(NCORES, N, NCLS)
    out = _call_tc(
        _tc_tail, jax.ShapeDtypeStruct((NGRAPH, NCLS), jnp.float32),
        acc3, hs3, dinv, b3.reshape(1, NCLS), batch.reshape(1, N))
    return out


# speedup vs baseline: 39.4113x; 1.6873x over previous
import functools

import jax
import jax.numpy as jnp
from jax import lax
from jax.experimental import pallas as pl
from jax.experimental.pallas import tpu as pltpu
from jax.experimental.pallas import tpu_sc as plsc

N = 10000
E = 320000
NGRAPH = 64
NCLS = 16
NCORES = 2
NSUB = 16
NW = NCORES * NSUB
CHUNK = 80
NCHUNK = (E // NW) // CHUNK
ROWS_SUB = N // NSUB
DEGW = 16

_mesh = plsc.VectorSubcoreMesh(core_axis_name="c", subcore_axis_name="s")
_sc_params = pltpu.CompilerParams(use_tc_tiling_on_sc=False)


@functools.partial(
    pl.kernel,
    out_type=jax.ShapeDtypeStruct((NCORES, NSUB, ROWS_SUB, DEGW), jnp.float32),
    mesh=_mesh,
    compiler_params=_sc_params,
    scratch_types=[
        pltpu.VMEM((NCHUNK, CHUNK), jnp.int32),
        pltpu.VMEM((CHUNK, DEGW), jnp.float32),
        pltpu.VMEM_SHARED((N, DEGW), jnp.float32),
    ],
)
def _deg_kernel(dst_hbm, ones_hbm, zeros_hbm, out_hbm, dst_v, ones_v, acc):
    c = lax.axis_index("c")
    s = lax.axis_index("s")
    wid = c * NSUB + s
    pltpu.sync_copy(zeros_hbm, acc.at[pl.ds(s * ROWS_SUB, ROWS_SUB)])
    pltpu.sync_copy(ones_hbm, ones_v)
    pltpu.sync_copy(dst_hbm.at[wid], dst_v)
    plsc.subcore_barrier()

    def body(i, carry):
        pltpu.sync_copy(ones_v, acc.at[dst_v.at[i]], add=True)
        return carry

    lax.fori_loop(0, NCHUNK, body, 0)
    plsc.subcore_barrier()
    pltpu.sync_copy(acc.at[pl.ds(s * ROWS_SUB, ROWS_SUB)], out_hbm.at[c, s])


NBUF = 5
NGROUP = NCHUNK // NBUF


def _make_agg(F):
    @functools.partial(
        pl.kernel,
        out_type=jax.ShapeDtypeStruct((NCORES, NSUB, ROWS_SUB, F), jnp.float32),
        mesh=_mesh,
        compiler_params=_sc_params,
        scratch_types=[
            pltpu.VMEM((NCHUNK, CHUNK), jnp.int32),
            pltpu.VMEM((NCHUNK, CHUNK), jnp.int32),
            pltpu.VMEM((NBUF, CHUNK, F), jnp.float32),
            pltpu.SemaphoreType.DMA((NBUF,)),
            pltpu.VMEM_SHARED((N, F), jnp.float32),
        ],
    )
    def agg(hs_hbm, src_hbm, dst_hbm, zeros_hbm, out_hbm,
            src_v, dst_v, rows_v, sems, acc):
        c = lax.axis_index("c")
        s = lax.axis_index("s")
        wid = c * NSUB + s
        pltpu.sync_copy(zeros_hbm, acc.at[pl.ds(s * ROWS_SUB, ROWS_SUB)])
        pltpu.sync_copy(src_hbm.at[wid], src_v)
        pltpu.sync_copy(dst_hbm.at[wid], dst_v)
        plsc.subcore_barrier()

        def group(g, carry):
            base = g * NBUF
            gcps = [
                pltpu.async_copy(hs_hbm.at[src_v.at[base + b]],
                                 rows_v.at[b], sems.at[b])
                for b in range(NBUF)
            ]
            scps = []
            for b in range(NBUF):
                gcps[b].wait()
                scps.append(
                    pltpu.async_copy(rows_v.at[b], acc.at[dst_v.at[base + b]],
                                     sems.at[b], add=True))
            for b in range(NBUF):
                scps[b].wait()
            return carry

        lax.fori_loop(0, NGROUP, group, 0)
        plsc.subcore_barrier()
        pltpu.sync_copy(acc.at[pl.ds(s * ROWS_SUB, ROWS_SUB)], out_hbm.at[c, s])

    return agg


_AGG = {F: _make_agg(F) for F in (32, 64, 16)}


def _tc_head(parts_ref, x_ref, w_ref, hs_ref, dinv_ref):
    deg = (parts_ref[0] + parts_ref[1])[:, 0:1] + 1.0
    dinv = lax.rsqrt(deg)
    p = jnp.dot(x_ref[...], w_ref[...], preferred_element_type=jnp.float32)
    hs_ref[...] = p * dinv
    dinv_ref[...] = dinv


def _tc_mid(parts_ref, hs_ref, dinv_ref, b_ref, w_ref, out_ref):
    dinv = dinv_ref[...]
    h = dinv * (parts_ref[0] + parts_ref[1] + hs_ref[...]) + b_ref[...]
    h = jnp.maximum(h, 0.0)
    out_ref[...] = jnp.dot(h, w_ref[...],
                           preferred_element_type=jnp.float32) * dinv


def _tc_tail(parts_ref, hs_ref, dinv_ref, b_ref, batch_ref, out_ref):
    dinv = dinv_ref[...]
    h3 = dinv * (parts_ref[0] + parts_ref[1] + hs_ref[...]) + b_ref[...]
    gids = lax.broadcasted_iota(jnp.int32, (NGRAPH, N), 0)
    sel = (batch_ref[...] == gids).astype(jnp.float32)
    sums = jnp.dot(sel, h3, preferred_element_type=jnp.float32)
    counts = jnp.sum(sel, axis=1, keepdims=True)
    pooled = sums / jnp.maximum(counts, 1.0)
    m = jnp.max(pooled, axis=1, keepdims=True)
    lse = jnp.log(jnp.sum(jnp.exp(pooled - m), axis=1, keepdims=True)) + m
    out_ref[...] = pooled - lse


def _call_tc(body, out_shapes, *args):
    return pl.pallas_call(
        body,
        out_shape=out_shapes,
    )(*args)


@jax.jit
def kernel(x, edge_index, batch, W1, b1, W2, b2, W3, b3):
    src = edge_index[0].reshape(NW, NCHUNK, CHUNK)
    dst = edge_index[1].reshape(NW, NCHUNK, CHUNK)
    ones_deg = jnp.ones((CHUNK, DEGW), jnp.float32)
    zeros_deg = jnp.zeros((ROWS_SUB, DEGW), jnp.float32)

    deg_parts = _deg_kernel(dst, ones_deg, zeros_deg).reshape(NCORES, N, DEGW)

    hs1, dinv = _call_tc(
        _tc_head,
        [jax.ShapeDtypeStruct((N, 32), jnp.float32),
         jax.ShapeDtypeStruct((N, 1), jnp.float32)],
        deg_parts, x, W1)

    acc1 = _AGG[32](hs1, src, dst,
                    jnp.zeros((ROWS_SUB, 32), jnp.float32)).reshape(NCORES, N, 32)
    hs2 = _call_tc(
        _tc_mid, jax.ShapeDtypeStruct((N, 64), jnp.float32),
        acc1, hs1, dinv, b1.reshape(1, 32), W2)

    acc2 = _AGG[64](hs2, src, dst,
                    jnp.zeros((ROWS_SUB, 64), jnp.float32)).reshape(NCORES, N, 64)
    hs3 = _call_tc(
        _tc_mid, jax.ShapeDtypeStruct((N, NCLS), jnp.float32),
        acc2, hs2, dinv, b2.reshape(1, 64), W3)

    acc3 = _AGG[16](hs3, src, dst,
                    jnp.zeros((ROWS_SUB, NCLS), jnp.float32)).reshape(NCORES, N, NCLS)
    out = _call_tc(
        _tc_tail, jax.ShapeDtypeStruct((NGRAPH, NCLS), jnp.float32),
        acc3, hs3, dinv, b3.reshape(1, NCLS), batch.reshape(1, N))
    return out


# speedup vs baseline: 43.2861x; 1.0983x over previous
import functools

import jax
import jax.numpy as jnp
from jax import lax
from jax.experimental import pallas as pl
from jax.experimental.pallas import tpu as pltpu
from jax.experimental.pallas import tpu_sc as plsc

N = 10000
E = 320000
NGRAPH = 64
NCLS = 16
NCORES = 2
NSUB = 16
NW = NCORES * NSUB
CHUNK = 80
NCHUNK = (E // NW) // CHUNK
ROWS_SUB = N // NSUB
DEGW = 16

_mesh = plsc.VectorSubcoreMesh(core_axis_name="c", subcore_axis_name="s")
_sc_params = pltpu.CompilerParams(use_tc_tiling_on_sc=False)


@functools.partial(
    pl.kernel,
    out_type=jax.ShapeDtypeStruct((NCORES, NSUB, ROWS_SUB, DEGW), jnp.float32),
    mesh=_mesh,
    compiler_params=_sc_params,
    scratch_types=[
        pltpu.VMEM((NCHUNK, CHUNK), jnp.int32),
        pltpu.VMEM((CHUNK, DEGW), jnp.float32),
        pltpu.VMEM_SHARED((N, DEGW), jnp.float32),
        pltpu.SemaphoreType.DMA((5,)),
    ],
)
def _deg_kernel(dst_hbm, ones_hbm, zeros_hbm, out_hbm, dst_v, ones_v, acc,
                dsems):
    c = lax.axis_index("c")
    s = lax.axis_index("s")
    wid = c * NSUB + s
    pltpu.sync_copy(zeros_hbm, acc.at[pl.ds(s * ROWS_SUB, ROWS_SUB)])
    pltpu.sync_copy(ones_hbm, ones_v)
    pltpu.sync_copy(dst_hbm.at[wid], dst_v)
    plsc.subcore_barrier()

    def body(g, carry):
        base = g * 5
        cps = [
            pltpu.async_copy(ones_v, acc.at[dst_v.at[base + b]],
                             dsems.at[b], add=True)
            for b in range(5)
        ]
        for cp in cps:
            cp.wait()
        return carry

    lax.fori_loop(0, NCHUNK // 5, body, 0)
    plsc.subcore_barrier()
    pltpu.sync_copy(acc.at[pl.ds(s * ROWS_SUB, ROWS_SUB)], out_hbm.at[c, s])


NBUF_BY_F = {16: 25, 32: 25, 64: 5}


def _make_agg(F):
    NBUF = NBUF_BY_F[F]
    NGROUP = NCHUNK // NBUF

    @functools.partial(
        pl.kernel,
        out_type=jax.ShapeDtypeStruct((NCORES, NSUB, ROWS_SUB, F), jnp.float32),
        mesh=_mesh,
        compiler_params=_sc_params,
        scratch_types=[
            pltpu.VMEM((NCHUNK, CHUNK), jnp.int32),
            pltpu.VMEM((NCHUNK, CHUNK), jnp.int32),
            pltpu.VMEM((NBUF, CHUNK, F), jnp.float32),
            pltpu.SemaphoreType.DMA((NBUF,)),
            pltpu.VMEM_SHARED((N, F), jnp.float32),
        ],
    )
    def agg(hs_hbm, src_hbm, dst_hbm, zeros_hbm, out_hbm,
            src_v, dst_v, rows_v, sems, acc):
        c = lax.axis_index("c")
        s = lax.axis_index("s")
        wid = c * NSUB + s
        pltpu.sync_copy(zeros_hbm, acc.at[pl.ds(s * ROWS_SUB, ROWS_SUB)])
        pltpu.sync_copy(src_hbm.at[wid], src_v)
        pltpu.sync_copy(dst_hbm.at[wid], dst_v)
        plsc.subcore_barrier()

        def group(g, carry):
            base = g * NBUF
            gcps = [
                pltpu.async_copy(hs_hbm.at[src_v.at[base + b]],
                                 rows_v.at[b], sems.at[b])
                for b in range(NBUF)
            ]
            scps = []
            for b in range(NBUF):
                gcps[b].wait()
                scps.append(
                    pltpu.async_copy(rows_v.at[b], acc.at[dst_v.at[base + b]],
                                     sems.at[b], add=True))
            for b in range(NBUF):
                scps[b].wait()
            return carry

        lax.fori_loop(0, NGROUP, group, 0)
        plsc.subcore_barrier()
        pltpu.sync_copy(acc.at[pl.ds(s * ROWS_SUB, ROWS_SUB)], out_hbm.at[c, s])

    return agg


_AGG = {F: _make_agg(F) for F in (32, 64, 16)}


def _tc_head(parts_ref, x_ref, w_ref, hs_ref, dinv_ref):
    deg = (parts_ref[0] + parts_ref[1])[:, 0:1] + 1.0
    dinv = lax.rsqrt(deg)
    p = jnp.dot(x_ref[...], w_ref[...], preferred_element_type=jnp.float32)
    hs_ref[...] = p * dinv
    dinv_ref[...] = dinv


def _tc_mid(parts_ref, hs_ref, dinv_ref, b_ref, w_ref, out_ref):
    dinv = dinv_ref[...]
    h = dinv * (parts_ref[0] + parts_ref[1] + hs_ref[...]) + b_ref[...]
    h = jnp.maximum(h, 0.0)
    out_ref[...] = jnp.dot(h, w_ref[...],
                           preferred_element_type=jnp.float32) * dinv


def _tc_tail(parts_ref, hs_ref, dinv_ref, b_ref, batch_ref, out_ref):
    dinv = dinv_ref[...]
    h3 = dinv * (parts_ref[0] + parts_ref[1] + hs_ref[...]) + b_ref[...]
    gids = lax.broadcasted_iota(jnp.int32, (NGRAPH, N), 0)
    sel = (batch_ref[...] == gids).astype(jnp.float32)
    sums = jnp.dot(sel, h3, preferred_element_type=jnp.float32)
    counts = jnp.sum(sel, axis=1, keepdims=True)
    pooled = sums / jnp.maximum(counts, 1.0)
    m = jnp.max(pooled, axis=1, keepdims=True)
    lse = jnp.log(jnp.sum(jnp.exp(pooled - m), axis=1, keepdims=True)) + m
    out_ref[...] = pooled - lse


def _call_tc(body, out_shapes, *args):
    return pl.pallas_call(
        body,
        out_shape=out_shapes,
    )(*args)


@jax.jit
def kernel(x, edge_index, batch, W1, b1, W2, b2, W3, b3):
    src = edge_index[0].reshape(NW, NCHUNK, CHUNK)
    dst = edge_index[1].reshape(NW, NCHUNK, CHUNK)
    ones_deg = jnp.ones((CHUNK, DEGW), jnp.float32)
    zeros_deg = jnp.zeros((ROWS_SUB, DEGW), jnp.float32)

    deg_parts = _deg_kernel(dst, ones_deg, zeros_deg).reshape(NCORES, N, DEGW)

    hs1, dinv = _call_tc(
        _tc_head,
        [jax.ShapeDtypeStruct((N, 32), jnp.float32),
         jax.ShapeDtypeStruct((N, 1), jnp.float32)],
        deg_parts, x, W1)

    acc1 = _AGG[32](hs1, src, dst,
                    jnp.zeros((ROWS_SUB, 32), jnp.float32)).reshape(NCORES, N, 32)
    hs2 = _call_tc(
        _tc_mid, jax.ShapeDtypeStruct((N, 64), jnp.float32),
        acc1, hs1, dinv, b1.reshape(1, 32), W2)

    acc2 = _AGG[64](hs2, src, dst,
                    jnp.zeros((ROWS_SUB, 64), jnp.float32)).reshape(NCORES, N, 64)
    hs3 = _call_tc(
        _tc_mid, jax.ShapeDtypeStruct((N, NCLS), jnp.float32),
        acc2, hs2, dinv, b2.reshape(1, 64), W3)

    acc3 = _AGG[16](hs3, src, dst,
                    jnp.zeros((ROWS_SUB, NCLS), jnp.float32)).reshape(NCORES, N, NCLS)
    out = _call_tc(
        _tc_tail, jax.ShapeDtypeStruct((NGRAPH, NCLS), jnp.float32),
        acc3, hs3, dinv, b3.reshape(1, NCLS), batch.reshape(1, N))
    return out


# speedup vs baseline: 44.8967x; 1.0372x over previous
import functools

import jax
import jax.numpy as jnp
from jax import lax
from jax.experimental import pallas as pl
from jax.experimental.pallas import tpu as pltpu
from jax.experimental.pallas import tpu_sc as plsc

N = 10000
E = 320000
NGRAPH = 64
NCLS = 16
NCORES = 2
NSUB = 16
NW = NCORES * NSUB
CHUNK = 80
NCHUNK = (E // NW) // CHUNK
ROWS_SUB = N // NSUB
DEGW = 16

_mesh = plsc.VectorSubcoreMesh(core_axis_name="c", subcore_axis_name="s")
_sc_params = pltpu.CompilerParams(use_tc_tiling_on_sc=False)


@functools.partial(
    pl.kernel,
    out_type=jax.ShapeDtypeStruct((NCORES, N, DEGW), jnp.float32),
    mesh=_mesh,
    compiler_params=_sc_params,
    scratch_types=[
        pltpu.VMEM((NCHUNK, CHUNK), jnp.int32),
        pltpu.VMEM((CHUNK, DEGW), jnp.float32),
        pltpu.VMEM_SHARED((N, DEGW), jnp.float32),
        pltpu.SemaphoreType.DMA((5,)),
    ],
)
def _deg_kernel(edge_hbm, ones_hbm, zeros_hbm, out_hbm, dst_v, ones_v, acc,
                dsems):
    c = lax.axis_index("c")
    s = lax.axis_index("s")
    wid = c * NSUB + s
    pltpu.sync_copy(zeros_hbm, acc.at[pl.ds(s * ROWS_SUB, ROWS_SUB)])
    pltpu.sync_copy(ones_hbm, ones_v)
    pltpu.sync_copy(edge_hbm.at[1, wid], dst_v)
    plsc.subcore_barrier()

    def body(g, carry):
        base = g * 5
        cps = [
            pltpu.async_copy(ones_v, acc.at[dst_v.at[base + b]],
                             dsems.at[b], add=True)
            for b in range(5)
        ]
        for cp in cps:
            cp.wait()
        return carry

    lax.fori_loop(0, NCHUNK // 5, body, 0)
    plsc.subcore_barrier()
    pltpu.sync_copy(acc.at[pl.ds(s * ROWS_SUB, ROWS_SUB)],
                    out_hbm.at[c, pl.ds(s * ROWS_SUB, ROWS_SUB)])


NBUF_BY_F = {16: 25, 32: 25, 64: 5}


def _make_agg(F):
    NBUF = NBUF_BY_F[F]
    NGROUP = NCHUNK // NBUF

    @functools.partial(
        pl.kernel,
        out_type=jax.ShapeDtypeStruct((NCORES, N, F), jnp.float32),
        mesh=_mesh,
        compiler_params=_sc_params,
        scratch_types=[
            pltpu.VMEM((NCHUNK, CHUNK), jnp.int32),
            pltpu.VMEM((NCHUNK, CHUNK), jnp.int32),
            pltpu.VMEM((NBUF, CHUNK, F), jnp.float32),
            pltpu.SemaphoreType.DMA((NBUF,)),
            pltpu.VMEM_SHARED((N, F), jnp.float32),
        ],
    )
    def agg(hs_hbm, edge_hbm, zeros_hbm, out_hbm,
            src_v, dst_v, rows_v, sems, acc):
        c = lax.axis_index("c")
        s = lax.axis_index("s")
        wid = c * NSUB + s
        pltpu.sync_copy(zeros_hbm, acc.at[pl.ds(s * ROWS_SUB, ROWS_SUB)])
        pltpu.sync_copy(edge_hbm.at[0, wid], src_v)
        pltpu.sync_copy(edge_hbm.at[1, wid], dst_v)
        plsc.subcore_barrier()

        def group(g, carry):
            base = g * NBUF
            gcps = [
                pltpu.async_copy(hs_hbm.at[src_v.at[base + b]],
                                 rows_v.at[b], sems.at[b])
                for b in range(NBUF)
            ]
            scps = []
            for b in range(NBUF):
                gcps[b].wait()
                scps.append(
                    pltpu.async_copy(rows_v.at[b], acc.at[dst_v.at[base + b]],
                                     sems.at[b], add=True))
            for b in range(NBUF):
                scps[b].wait()
            return carry

        lax.fori_loop(0, NGROUP, group, 0)
        plsc.subcore_barrier()
        pltpu.sync_copy(acc.at[pl.ds(s * ROWS_SUB, ROWS_SUB)],
                        out_hbm.at[c, pl.ds(s * ROWS_SUB, ROWS_SUB)])

    return agg


_AGG = {F: _make_agg(F) for F in (32, 64, 16)}


def _tc_mm(x_ref, w_ref, out_ref):
    out_ref[...] = jnp.dot(x_ref[...], w_ref[...],
                           preferred_element_type=jnp.float32)


def _tc_scale(parts_ref, p_ref, hs_ref, dinv_ref):
    deg = (parts_ref[0] + parts_ref[1])[:, 0:1] + 1.0
    dinv = lax.rsqrt(deg)
    hs_ref[...] = p_ref[...] * dinv
    dinv_ref[...] = dinv


def _tc_mid(parts_ref, hs_ref, dinv_ref, b_ref, w_ref, out_ref):
    dinv = dinv_ref[...]
    h = dinv * (parts_ref[0] + parts_ref[1] + hs_ref[...]) + b_ref[...]
    h = jnp.maximum(h, 0.0)
    out_ref[...] = jnp.dot(h, w_ref[...],
                           preferred_element_type=jnp.float32) * dinv


def _tc_tail(parts_ref, hs_ref, dinv_ref, b_ref, batch_ref, out_ref):
    dinv = dinv_ref[...]
    h3 = dinv * (parts_ref[0] + parts_ref[1] + hs_ref[...]) + b_ref[...]
    gids = lax.broadcasted_iota(jnp.int32, (NGRAPH, N), 0)
    sel = (batch_ref[...] == gids).astype(jnp.float32)
    sums = jnp.dot(sel, h3, preferred_element_type=jnp.float32)
    counts = jnp.sum(sel, axis=1, keepdims=True)
    pooled = sums / jnp.maximum(counts, 1.0)
    m = jnp.max(pooled, axis=1, keepdims=True)
    lse = jnp.log(jnp.sum(jnp.exp(pooled - m), axis=1, keepdims=True)) + m
    out_ref[...] = pooled - lse


def _call_tc(body, out_shapes, *args):
    return pl.pallas_call(
        body,
        out_shape=out_shapes,
    )(*args)


@jax.jit
def kernel(x, edge_index, batch, W1, b1, W2, b2, W3, b3):
    edges = edge_index.reshape(2, NW, NCHUNK, CHUNK)
    ones_deg = jnp.ones((CHUNK, DEGW), jnp.float32)
    zeros_deg = jnp.zeros((ROWS_SUB, DEGW), jnp.float32)

    deg_parts = _deg_kernel(edges, ones_deg, zeros_deg)
    p1 = _call_tc(_tc_mm, jax.ShapeDtypeStruct((N, 32), jnp.float32), x, W1)
    hs1, dinv = _call_tc(
        _tc_scale,
        [jax.ShapeDtypeStruct((N, 32), jnp.float32),
         jax.ShapeDtypeStruct((N, 1), jnp.float32)],
        deg_parts, p1)

    acc1 = _AGG[32](hs1, edges, jnp.zeros((ROWS_SUB, 32), jnp.float32))
    hs2 = _call_tc(
        _tc_mid, jax.ShapeDtypeStruct((N, 64), jnp.float32),
        acc1, hs1, dinv, b1.reshape(1, 32), W2)

    acc2 = _AGG[64](hs2, edges, jnp.zeros((ROWS_SUB, 64), jnp.float32))
    hs3 = _call_tc(
        _tc_mid, jax.ShapeDtypeStruct((N, NCLS), jnp.float32),
        acc2, hs2, dinv, b2.reshape(1, 64), W3)

    acc3 = _AGG[16](hs3, edges, jnp.zeros((ROWS_SUB, NCLS), jnp.float32))
    out = _call_tc(
        _tc_tail, jax.ShapeDtypeStruct((NGRAPH, NCLS), jnp.float32),
        acc3, hs3, dinv, b3.reshape(1, NCLS), batch.reshape(1, N))
    return out


# speedup vs baseline: 49.2798x; 1.0976x over previous
import functools

import jax
import jax.numpy as jnp
from jax import lax
from jax.experimental import pallas as pl
from jax.experimental.pallas import tpu as pltpu
from jax.experimental.pallas import tpu_sc as plsc

N = 10000
E = 320000
NGRAPH = 64
NCLS = 16
NCORES = 2
NSUB = 16
NW = NCORES * NSUB
CHUNK = 80
NCHUNK = (E // NW) // CHUNK
ROWS_SUB = N // NSUB
DEGW = 16

_mesh = plsc.VectorSubcoreMesh(core_axis_name="c", subcore_axis_name="s")
_sc_params = pltpu.CompilerParams(use_tc_tiling_on_sc=False)


@functools.partial(
    pl.kernel,
    out_type=jax.ShapeDtypeStruct((NCORES, N, 128), jnp.float32),
    mesh=_mesh,
    compiler_params=_sc_params,
    scratch_types=[
        pltpu.VMEM((NCHUNK, CHUNK), jnp.int32),
        pltpu.VMEM((CHUNK, DEGW), jnp.float32),
        pltpu.VMEM_SHARED((N, DEGW), jnp.float32),
        pltpu.SemaphoreType.DMA((5,)),
    ],
)
def _deg_kernel(edge_hbm, ones_hbm, zeros_hbm, out_hbm, dst_v, ones_v, acc,
                dsems):
    c = lax.axis_index("c")
    s = lax.axis_index("s")
    wid = c * NSUB + s
    pltpu.sync_copy(zeros_hbm, acc.at[pl.ds(s * ROWS_SUB, ROWS_SUB)])
    pltpu.sync_copy(ones_hbm, ones_v)
    pltpu.sync_copy(edge_hbm.at[1, wid], dst_v)
    plsc.subcore_barrier()

    def body(g, carry):
        base = g * 5
        cps = [
            pltpu.async_copy(ones_v, acc.at[dst_v.at[base + b]],
                             dsems.at[b], add=True)
            for b in range(5)
        ]
        for cp in cps:
            cp.wait()
        return carry

    lax.fori_loop(0, NCHUNK // 5, body, 0)
    plsc.subcore_barrier()
    pltpu.sync_copy(acc.at[pl.ds(s * ROWS_SUB, ROWS_SUB)],
                    out_hbm.at[c, pl.ds(s * ROWS_SUB, ROWS_SUB), pl.ds(0, DEGW)])


NBUF_BY_F = {16: 25, 32: 25, 64: 5}


def _make_agg(F):
    NBUF = NBUF_BY_F[F]
    NGROUP = NCHUNK // NBUF

    @functools.partial(
        pl.kernel,
        out_type=jax.ShapeDtypeStruct((NCORES, N, 128), jnp.float32),
        mesh=_mesh,
        compiler_params=_sc_params,
        scratch_types=[
            pltpu.VMEM((NCHUNK, CHUNK), jnp.int32),
            pltpu.VMEM((NCHUNK, CHUNK), jnp.int32),
            pltpu.VMEM((NBUF, CHUNK, F), jnp.float32),
            pltpu.SemaphoreType.DMA((NBUF,)),
            pltpu.VMEM_SHARED((N, F), jnp.float32),
        ],
    )
    def agg(hs_hbm, edge_hbm, zeros_hbm, out_hbm,
            src_v, dst_v, rows_v, sems, acc):
        c = lax.axis_index("c")
        s = lax.axis_index("s")
        wid = c * NSUB + s
        pltpu.sync_copy(zeros_hbm, acc.at[pl.ds(s * ROWS_SUB, ROWS_SUB)])
        pltpu.sync_copy(edge_hbm.at[0, wid], src_v)
        pltpu.sync_copy(edge_hbm.at[1, wid], dst_v)
        plsc.subcore_barrier()

        def group(g, carry):
            base = g * NBUF
            gcps = [
                pltpu.async_copy(hs_hbm.at[src_v.at[base + b]],
                                 rows_v.at[b], sems.at[b])
                for b in range(NBUF)
            ]
            scps = []
            for b in range(NBUF):
                gcps[b].wait()
                scps.append(
                    pltpu.async_copy(rows_v.at[b], acc.at[dst_v.at[base + b]],
                                     sems.at[b], add=True))
            for b in range(NBUF):
                scps[b].wait()
            return carry

        lax.fori_loop(0, NGROUP, group, 0)
        plsc.subcore_barrier()
        pltpu.sync_copy(acc.at[pl.ds(s * ROWS_SUB, ROWS_SUB)],
                        out_hbm.at[c, pl.ds(s * ROWS_SUB, ROWS_SUB),
                                   pl.ds(0, F)])

    return agg


_AGG = {F: _make_agg(F) for F in (32, 64, 16)}


def _tc_mm(x_ref, w_ref, out_ref):
    out_ref[...] = jnp.dot(x_ref[...], w_ref[...],
                           preferred_element_type=jnp.float32)


def _tc_scale(parts_ref, p_ref, hs_ref, dinv_ref):
    deg = parts_ref[0, :, 0:1] + parts_ref[1, :, 0:1] + 1.0
    dinv = lax.rsqrt(deg)
    hs_ref[...] = p_ref[...] * dinv
    dinv_ref[...] = dinv


def _tc_mid(parts_ref, hs_ref, dinv_ref, b_ref, w_ref, out_ref):
    dinv = dinv_ref[...]
    F = hs_ref.shape[1]
    h = dinv * (parts_ref[0, :, 0:F] + parts_ref[1, :, 0:F]
                + hs_ref[...]) + b_ref[...]
    h = jnp.maximum(h, 0.0)
    out_ref[...] = jnp.dot(h, w_ref[...],
                           preferred_element_type=jnp.float32) * dinv


def _tc_tail(parts_ref, hs_ref, dinv_ref, b_ref, batch_ref, out_ref):
    dinv = dinv_ref[...]
    h3 = dinv * (parts_ref[0, :, 0:NCLS] + parts_ref[1, :, 0:NCLS]
                 + hs_ref[...]) + b_ref[...]
    gids = lax.broadcasted_iota(jnp.int32, (NGRAPH, N), 0)
    sel = (batch_ref[...] == gids).astype(jnp.float32)
    sums = jnp.dot(sel, h3, preferred_element_type=jnp.float32)
    counts = jnp.sum(sel, axis=1, keepdims=True)
    pooled = sums / jnp.maximum(counts, 1.0)
    m = jnp.max(pooled, axis=1, keepdims=True)
    lse = jnp.log(jnp.sum(jnp.exp(pooled - m), axis=1, keepdims=True)) + m
    out_ref[...] = pooled - lse


def _call_tc(body, out_shapes, *args):
    return pl.pallas_call(
        body,
        out_shape=out_shapes,
    )(*args)


@jax.jit
def kernel(x, edge_index, batch, W1, b1, W2, b2, W3, b3):
    edges = edge_index.reshape(2, NW, NCHUNK, CHUNK)
    ones_deg = jnp.ones((CHUNK, DEGW), jnp.float32)
    zeros_deg = jnp.zeros((ROWS_SUB, DEGW), jnp.float32)

    deg_parts = _deg_kernel(edges, ones_deg, zeros_deg)
    p1 = _call_tc(_tc_mm, jax.ShapeDtypeStruct((N, 32), jnp.float32), x, W1)
    hs1, dinv = _call_tc(
        _tc_scale,
        [jax.ShapeDtypeStruct((N, 32), jnp.float32),
         jax.ShapeDtypeStruct((N, 1), jnp.float32)],
        deg_parts, p1)

    acc1 = _AGG[32](hs1, edges, jnp.zeros((ROWS_SUB, 32), jnp.float32))
    hs2 = _call_tc(
        _tc_mid, jax.ShapeDtypeStruct((N, 64), jnp.float32),
        acc1, hs1, dinv, b1.reshape(1, 32), W2)

    acc2 = _AGG[64](hs2, edges, jnp.zeros((ROWS_SUB, 64), jnp.float32))
    hs3 = _call_tc(
        _tc_mid, jax.ShapeDtypeStruct((N, NCLS), jnp.float32),
        acc2, hs2, dinv, b2.reshape(1, 64), W3)

    acc3 = _AGG[16](hs3, edges, jnp.zeros((ROWS_SUB, NCLS), jnp.float32))
    out = _call_tc(
        _tc_tail, jax.ShapeDtypeStruct((NGRAPH, NCLS), jnp.float32),
        acc3, hs3, dinv, b3.reshape(1, NCLS), batch.reshape(1, N))
    return out


# speedup vs baseline: 49.5584x; 1.0057x over previous
import functools

import jax
import jax.numpy as jnp
from jax import lax
from jax.experimental import pallas as pl
from jax.experimental.pallas import tpu as pltpu
from jax.experimental.pallas import tpu_sc as plsc

N = 10000
E = 320000
NGRAPH = 64
NCLS = 16
NCORES = 2
NSUB = 16
NW = NCORES * NSUB
CHUNK = 80
NCHUNK = (E // NW) // CHUNK
ROWS_SUB = N // NSUB
DEGW = 16

_mesh = plsc.VectorSubcoreMesh(core_axis_name="c", subcore_axis_name="s")
_sc_params = pltpu.CompilerParams(use_tc_tiling_on_sc=False)


@functools.partial(
    pl.kernel,
    out_type=jax.ShapeDtypeStruct((NCORES, N, 128), jnp.float32),
    mesh=_mesh,
    compiler_params=_sc_params,
    scratch_types=[
        pltpu.VMEM((NCHUNK, CHUNK), jnp.int32),
        pltpu.VMEM((CHUNK, DEGW), jnp.float32),
        pltpu.VMEM_SHARED((N, DEGW), jnp.float32),
        pltpu.SemaphoreType.DMA((5,)),
    ],
)
def _deg_kernel(edge_hbm, ones_hbm, zeros_hbm, out_hbm, dst_v, ones_v, acc,
                dsems):
    c = lax.axis_index("c")
    s = lax.axis_index("s")
    wid = c * NSUB + s
    pltpu.sync_copy(zeros_hbm, acc.at[pl.ds(s * ROWS_SUB, ROWS_SUB)])
    pltpu.sync_copy(ones_hbm, ones_v)
    pltpu.sync_copy(edge_hbm.at[1, wid], dst_v)
    plsc.subcore_barrier()

    def body(g, carry):
        base = g * 5
        cps = [
            pltpu.async_copy(ones_v, acc.at[dst_v.at[base + b]],
                             dsems.at[b], add=True)
            for b in range(5)
        ]
        for cp in cps:
            cp.wait()
        return carry

    lax.fori_loop(0, NCHUNK // 5, body, 0)
    plsc.subcore_barrier()
    pltpu.sync_copy(acc.at[pl.ds(s * ROWS_SUB, ROWS_SUB)],
                    out_hbm.at[c, pl.ds(s * ROWS_SUB, ROWS_SUB), pl.ds(0, DEGW)])


NBUF_BY_F = {16: 25, 32: 25, 64: 5}


def _make_agg(F):
    NBUF = NBUF_BY_F[F]
    NGROUP = NCHUNK // NBUF

    @functools.partial(
        pl.kernel,
        out_type=jax.ShapeDtypeStruct((NCORES, N, 128), jnp.float32),
        mesh=_mesh,
        compiler_params=_sc_params,
        scratch_types=[
            pltpu.VMEM((NCHUNK, CHUNK), jnp.int32),
            pltpu.VMEM((NCHUNK, CHUNK), jnp.int32),
            pltpu.VMEM((NBUF, CHUNK, F), jnp.float32),
            pltpu.SemaphoreType.DMA((NBUF,)),
            pltpu.VMEM_SHARED((N, F), jnp.float32),
        ],
    )
    def agg(hs_hbm, edge_hbm, zeros_hbm, out_hbm,
            src_v, dst_v, rows_v, sems, acc):
        c = lax.axis_index("c")
        s = lax.axis_index("s")
        wid = c * NSUB + s
        pltpu.sync_copy(zeros_hbm, acc.at[pl.ds(s * ROWS_SUB, ROWS_SUB)])
        pltpu.sync_copy(edge_hbm.at[0, wid], src_v)
        pltpu.sync_copy(edge_hbm.at[1, wid], dst_v)
        plsc.subcore_barrier()

        def group(g, carry):
            base = g * NBUF
            gcps = [
                pltpu.async_copy(hs_hbm.at[src_v.at[base + b]],
                                 rows_v.at[b], sems.at[b])
                for b in range(NBUF)
            ]
            scps = []
            for b in range(NBUF):
                gcps[b].wait()
                scps.append(
                    pltpu.async_copy(rows_v.at[b], acc.at[dst_v.at[base + b]],
                                     sems.at[b], add=True))
            for b in range(NBUF):
                scps[b].wait()
            return carry

        lax.fori_loop(0, NGROUP, group, 0)
        plsc.subcore_barrier()
        pltpu.sync_copy(acc.at[pl.ds(s * ROWS_SUB, ROWS_SUB)],
                        out_hbm.at[c, pl.ds(s * ROWS_SUB, ROWS_SUB),
                                   pl.ds(0, F)])

    return agg


_AGG = {F: _make_agg(F) for F in (32, 64, 16)}


def _tc_mm(x_ref, w_ref, out_ref):
    out_ref[...] = jnp.dot(x_ref[...], w_ref[...],
                           preferred_element_type=jnp.float32)


def _tc_scale(parts_ref, p_ref, hs_ref, dinv_ref):
    deg = parts_ref[0, :, 0:1] + parts_ref[1, :, 0:1] + 1.0
    dinv = lax.rsqrt(deg)
    hs_ref[...] = p_ref[...] * dinv
    dinv_ref[...] = dinv


def _tc_mid(parts_ref, hs_ref, dinv_ref, b_ref, w_ref, out_ref):
    dinv = dinv_ref[...]
    F = hs_ref.shape[1]
    h = dinv * (parts_ref[0, :, 0:F] + parts_ref[1, :, 0:F]
                + hs_ref[...]) + b_ref[...]
    h = jnp.maximum(h, 0.0)
    out_ref[...] = jnp.dot(h, w_ref[...],
                           preferred_element_type=jnp.float32) * dinv


def _tc_tail(parts_ref, hs_ref, dinv_ref, b_ref, batch_ref, out_ref):
    dinv = dinv_ref[...]
    h3 = dinv * (parts_ref[0, :, 0:NCLS] + parts_ref[1, :, 0:NCLS]
                 + hs_ref[...]) + b_ref[...]
    gids = lax.broadcasted_iota(jnp.int32, (NGRAPH, N), 0)
    sel = (batch_ref[...] == gids).astype(jnp.float32)
    sums = jnp.dot(sel, h3, preferred_element_type=jnp.float32)
    counts = jnp.sum(sel, axis=1, keepdims=True)
    pooled = sums / jnp.maximum(counts, 1.0)
    m = jnp.max(pooled, axis=1, keepdims=True)
    lse = jnp.log(jnp.sum(jnp.exp(pooled - m), axis=1, keepdims=True)) + m
    out_ref[...] = pooled - lse


def _call_tc(body, out_shapes, *args):
    return pl.pallas_call(
        body,
        out_shape=out_shapes,
    )(*args)


_RB = 2000
_GRID = N // _RB


def _rows_spec(shape):
    if len(shape) == 3:
        return pl.BlockSpec((shape[0], _RB, shape[2]), lambda i: (0, i, 0))
    return pl.BlockSpec((_RB, shape[1]), lambda i: (i, 0))


def _full_spec(shape):
    return pl.BlockSpec(shape, lambda i: tuple(0 for _ in shape))


def _call_tc_rows(body, out_shapes, row_args, full_args):
    flat_outs = out_shapes if isinstance(out_shapes, list) else [out_shapes]
    in_specs = ([_rows_spec(a.shape) for a in row_args]
                + [_full_spec(a.shape) for a in full_args])
    out_specs = [_rows_spec(o.shape) for o in flat_outs]
    res = pl.pallas_call(
        body,
        grid=(_GRID,),
        in_specs=in_specs,
        out_specs=out_specs if isinstance(out_shapes, list) else out_specs[0],
        out_shape=out_shapes,
    )(*row_args, *full_args)
    return res


@jax.jit
def kernel(x, edge_index, batch, W1, b1, W2, b2, W3, b3):
    edges = edge_index.reshape(2, NW, NCHUNK, CHUNK)
    ones_deg = jnp.ones((CHUNK, DEGW), jnp.float32)
    zeros_deg = jnp.zeros((ROWS_SUB, DEGW), jnp.float32)

    deg_parts = _deg_kernel(edges, ones_deg, zeros_deg)
    p1 = _call_tc_rows(_tc_mm, jax.ShapeDtypeStruct((N, 32), jnp.float32),
                       (x,), (W1,))
    hs1, dinv = _call_tc_rows(
        _tc_scale,
        [jax.ShapeDtypeStruct((N, 32), jnp.float32),
         jax.ShapeDtypeStruct((N, 1), jnp.float32)],
        (deg_parts, p1), ())

    acc1 = _AGG[32](hs1, edges, jnp.zeros((ROWS_SUB, 32), jnp.float32))
    hs2 = _call_tc_rows(
        _tc_mid, jax.ShapeDtypeStruct((N, 64), jnp.float32),
        (acc1, hs1, dinv), (b1.reshape(1, 32), W2))

    acc2 = _AGG[64](hs2, edges, jnp.zeros((ROWS_SUB, 64), jnp.float32))
    hs3 = _call_tc_rows(
        _tc_mid, jax.ShapeDtypeStruct((N, NCLS), jnp.float32),
        (acc2, hs2, dinv), (b2.reshape(1, 64), W3))

    acc3 = _AGG[16](hs3, edges, jnp.zeros((ROWS_SUB, NCLS), jnp.float32))
    out = _call_tc(
        _tc_tail, jax.ShapeDtypeStruct((NGRAPH, NCLS), jnp.float32),
        acc3, hs3, dinv, b3.reshape(1, NCLS), batch.reshape(1, N))
    return out
